# baseline (speedup 1.0000x reference)
import functools

import jax
import jax.numpy as jnp
from jax import lax
from jax.experimental import pallas as pl
from jax.experimental.pallas import tpu as pltpu
from jax.experimental.pallas import tpu_sc as plsc

_NC = 2
_NS = 16
_NW = _NC * _NS
_CHUNK = 128
_NBUF = 4
_QW = 32


@functools.cache
def _make_gather(n_rows, vocab):
    qv = vocab // 4
    assert n_rows % (_NW * _CHUNK) == 0
    n_chunks = n_rows // (_NW * _CHUNK)
    per_w = n_chunks * _CHUNK
    mesh = plsc.VectorSubcoreMesh(core_axis_name="c", subcore_axis_name="s")

    @functools.partial(
        pl.kernel,
        out_type=jax.ShapeDtypeStruct((n_rows, _QW), jnp.int32),
        mesh=mesh,
        scratch_types=[
            pltpu.VMEM((n_chunks, _CHUNK), jnp.int32),
            pltpu.VMEM((n_chunks, _CHUNK), jnp.int32),
            *[pltpu.VMEM((_CHUNK, 4 * _QW), jnp.int32) for _ in range(_NBUF)],
            pltpu.VMEM((_CHUNK, _QW), jnp.int32),
            pltpu.SemaphoreType.DMA,
            pltpu.SemaphoreType.DMA,
        ],
        compiler_params=pltpu.CompilerParams(
            use_tc_tiling_on_sc=False, needs_layout_passes=False
        ),
    )
    def gather(table_hbm, idx_hbm, out_hbm, idx_v, q_v, *rest):
        bufs = rest[:_NBUF]
        extr, g_sem, w_sem = rest[_NBUF], rest[_NBUF + 1], rest[_NBUF + 2]
        wid = lax.axis_index("s") * _NC + lax.axis_index("c")
        lanes = lax.broadcasted_iota(jnp.int32, (16,), 0)
        pltpu.sync_copy(idx_hbm.at[wid], idx_v)
        base = wid * per_w

        def split(c):
            for g in range(_CHUNK // 16):
                r = idx_v[c, pl.ds(g * 16, 16)]
                m = (
                    (r >= qv).astype(jnp.int32)
                    + (r >= 2 * qv).astype(jnp.int32)
                    + (r >= 3 * qv).astype(jnp.int32)
                )
                idx_v[c, pl.ds(g * 16, 16)] = r - m * qv
                q_v[c, pl.ds(g * 16, 16)] = m * _QW

        for b in range(_NBUF):
            split(b)
            pltpu.async_copy(table_hbm.at[idx_v.at[b]], bufs[b], g_sem)

        def body(step, carry):
            c0 = step * _NBUF
            for b in range(_NBUF):
                c = c0 + b
                pltpu.make_async_copy(
                    table_hbm.at[idx_v.at[c]], bufs[b], g_sem
                ).wait()
                for g in range(_CHUNK // 16):
                    rows = lanes + g * 16
                    qoff = q_v[c, pl.ds(g * 16, 16)]
                    zeros = lanes * 0
                    for q in range(_QW):
                        vals = plsc.load_gather(bufs[b], [rows, qoff + q])
                        plsc.store_scatter(extr, [rows, zeros + q], vals)
                pltpu.sync_copy(extr, out_hbm.at[pl.ds(base + c * _CHUNK, _CHUNK)])

                @pl.when(c + _NBUF < n_chunks)
                def _():
                    split(c + _NBUF)
                    pltpu.async_copy(
                        table_hbm.at[idx_v.at[c + _NBUF]], bufs[b], g_sem
                    )

            return carry

        lax.fori_loop(0, n_chunks // _NBUF, body, 0)

    return gather


def _pack_table(table):
    qv = table.shape[0] // 4
    u = jax.lax.bitcast_convert_type(table, jnp.uint32)
    r16 = (u + jnp.uint32(0x7FFF) + ((u >> 16) & jnp.uint32(1))) >> 16
    parts = [
        r16[m * qv : (m + 1) * qv, :32] | (r16[m * qv : (m + 1) * qv, 32:] << 16)
        for m in range(4)
    ]
    return jax.lax.bitcast_convert_type(
        jnp.concatenate(parts, axis=1), jnp.int32
    )


def _mlp_body(flat_ref, w1_ref, b1_ref, w2_ref, b2_ref, out_ref):
    w1 = w1_ref[...].astype(jnp.bfloat16)
    h = jnp.tanh(
        jnp.dot(flat_ref[...], w1, preferred_element_type=jnp.float32)
        + b1_ref[...]
    )
    w2 = w2_ref[...].astype(jnp.bfloat16)
    out_ref[...] = (
        jnp.dot(h.astype(jnp.bfloat16), w2, preferred_element_type=jnp.float32)
        + b2_ref[...]
    )


@functools.cache
def _make_mlp(batch, d_in, d_hidden, d_out, bm):
    grid = (batch // bm,)
    return pl.pallas_call(
        _mlp_body,
        grid=grid,
        in_specs=[
            pl.BlockSpec((bm, d_in), lambda i: (i, 0)),
            pl.BlockSpec((d_in, d_hidden), lambda i: (0, 0)),
            pl.BlockSpec((1, d_hidden), lambda i: (0, 0)),
            pl.BlockSpec((d_hidden, d_out), lambda i: (0, 0)),
            pl.BlockSpec((1, d_out), lambda i: (0, 0)),
        ],
        out_specs=pl.BlockSpec((bm, d_out), lambda i: (i, 0)),
        out_shape=jax.ShapeDtypeStruct((batch, d_out), jnp.float32),
    )


def kernel(x, table, W1, b1, W2, b2):
    batch, window = x.shape
    vocab, emb = table.shape
    n_rows = batch * window
    tbl = _pack_table(table)
    idx = x.astype(jnp.int32).reshape(_NW, n_rows // (_NW * _CHUNK), _CHUNK)
    gathered = _make_gather(n_rows, vocab)(tbl, idx)
    flat = jax.lax.bitcast_convert_type(gathered, jnp.bfloat16).reshape(
        batch, window * emb
    )
    w1p = (
        W1.reshape(window, 2, emb // 2, W1.shape[1])
        .transpose(0, 2, 1, 3)
        .reshape(window * emb, W1.shape[1])
    )
    mlp = _make_mlp(batch, window * emb, W1.shape[1], W2.shape[1], 2048)
    return mlp(flat, w1p, b1.reshape(1, -1), W2, b2.reshape(1, -1))


# speedup vs baseline: 5.8238x; 5.8238x over previous
import functools

import jax
import jax.numpy as jnp
from jax import lax
from jax.experimental import pallas as pl
from jax.experimental.pallas import tpu as pltpu
from jax.experimental.pallas import tpu_sc as plsc

_NC = 2
_NS = 16
_NW = _NC * _NS
_CHUNK = 128
_NBUF = 4
_PW = 128


@functools.cache
def _make_gather(n_rows):
    assert n_rows % (_NW * _CHUNK) == 0
    n_chunks = n_rows // (_NW * _CHUNK)
    per_w = n_chunks * _CHUNK
    mesh = plsc.VectorSubcoreMesh(core_axis_name="c", subcore_axis_name="s")

    @functools.partial(
        pl.kernel,
        out_type=jax.ShapeDtypeStruct((n_rows, _PW), jnp.float32),
        mesh=mesh,
        scratch_types=[
            pltpu.VMEM((n_chunks, _CHUNK), jnp.int32),
            *[pltpu.VMEM((_CHUNK, _PW), jnp.float32) for _ in range(_NBUF)],
            pltpu.SemaphoreType.DMA,
        ],
        compiler_params=pltpu.CompilerParams(
            use_tc_tiling_on_sc=False, needs_layout_passes=False
        ),
    )
    def gather(table_hbm, idx_hbm, out_hbm, idx_v, *rest):
        bufs, g_sem = rest[:_NBUF], rest[_NBUF]
        wid = lax.axis_index("s") * _NC + lax.axis_index("c")
        pltpu.sync_copy(idx_hbm.at[wid], idx_v)
        base = wid * per_w

        for b in range(_NBUF):
            pltpu.async_copy(table_hbm.at[idx_v.at[b]], bufs[b], g_sem)

        def body(step, carry):
            c0 = step * _NBUF
            for b in range(_NBUF):
                c = c0 + b
                pltpu.make_async_copy(
                    table_hbm.at[idx_v.at[c]], bufs[b], g_sem
                ).wait()
                pltpu.sync_copy(
                    bufs[b], out_hbm.at[pl.ds(base + c * _CHUNK, _CHUNK)]
                )

                @pl.when(c + _NBUF < n_chunks)
                def _():
                    pltpu.async_copy(
                        table_hbm.at[idx_v.at[c + _NBUF]], bufs[b], g_sem
                    )

            return carry

        lax.fori_loop(0, n_chunks // _NBUF, body, 0)

    return gather


def _mlp_body(flat_ref, w1_ref, b1_ref, w2_ref, b2_ref, out_ref):
    flat = flat_ref[...].astype(jnp.bfloat16)
    w1 = w1_ref[...].astype(jnp.bfloat16)
    h = jnp.tanh(
        jnp.dot(flat, w1, preferred_element_type=jnp.float32) + b1_ref[...]
    )
    w2 = w2_ref[...].astype(jnp.bfloat16)
    out_ref[...] = (
        jnp.dot(h.astype(jnp.bfloat16), w2, preferred_element_type=jnp.float32)
        + b2_ref[...]
    )


@functools.cache
def _make_mlp(batch, d_in, d_hidden, d_out, bm):
    grid = (batch // bm,)
    return pl.pallas_call(
        _mlp_body,
        grid=grid,
        in_specs=[
            pl.BlockSpec((bm, d_in), lambda i: (i, 0)),
            pl.BlockSpec((d_in, d_hidden), lambda i: (0, 0)),
            pl.BlockSpec((1, d_hidden), lambda i: (0, 0)),
            pl.BlockSpec((d_hidden, d_out), lambda i: (0, 0)),
            pl.BlockSpec((1, d_out), lambda i: (0, 0)),
        ],
        out_specs=pl.BlockSpec((bm, d_out), lambda i: (i, 0)),
        out_shape=jax.ShapeDtypeStruct((batch, d_out), jnp.float32),
    )


def kernel(x, table, W1, b1, W2, b2):
    batch, window = x.shape
    vocab, emb = table.shape
    hidden = W1.shape[1]
    n_rows = batch * window
    tbl = jnp.pad(table, ((0, 0), (0, _PW - emb)))
    idx = x.astype(jnp.int32).reshape(_NW, n_rows // (_NW * _CHUNK), _CHUNK)
    gathered = _make_gather(n_rows)(tbl, idx)
    flat = gathered.reshape(batch, window * _PW)
    w1p = jnp.concatenate(
        [
            W1.reshape(window, emb, hidden),
            jnp.zeros((window, _PW - emb, hidden), W1.dtype),
        ],
        axis=1,
    ).reshape(window * _PW, hidden)
    mlp = _make_mlp(batch, window * _PW, hidden, W2.shape[1], 2048)
    return mlp(flat, w1p, b1.reshape(1, -1), W2, b2.reshape(1, -1))


# speedup vs baseline: 5.8249x; 1.0002x over previous
import functools

import jax
import jax.numpy as jnp
from jax import lax
from jax.experimental import pallas as pl
from jax.experimental.pallas import tpu as pltpu
from jax.experimental.pallas import tpu_sc as plsc

_NC = 2
_NS = 16
_NW = _NC * _NS
_CHUNK = 128
_NBUF = 4
_PW = 128


@functools.cache
def _make_gather(n_rows):
    assert n_rows % (_NW * _CHUNK) == 0
    n_chunks = n_rows // (_NW * _CHUNK)
    per_w = n_chunks * _CHUNK
    mesh = plsc.VectorSubcoreMesh(core_axis_name="c", subcore_axis_name="s")

    @functools.partial(
        pl.kernel,
        out_type=jax.ShapeDtypeStruct((n_rows, _PW), jnp.float32),
        mesh=mesh,
        scratch_types=[
            pltpu.VMEM((n_chunks, _CHUNK), jnp.int32),
            *[pltpu.VMEM((_CHUNK, _PW), jnp.float32) for _ in range(_NBUF)],
            pltpu.SemaphoreType.DMA,
        ],
        compiler_params=pltpu.CompilerParams(
            use_tc_tiling_on_sc=False, needs_layout_passes=False
        ),
    )
    def gather(table_hbm, idx_hbm, out_hbm, idx_v, *rest):
        bufs, g_sem = rest[:_NBUF], rest[_NBUF]
        wid = lax.axis_index("s") * _NC + lax.axis_index("c")
        pltpu.sync_copy(idx_hbm.at[wid], idx_v)
        base = wid * per_w

        for b in range(_NBUF):
            pltpu.async_copy(table_hbm.at[idx_v.at[b]], bufs[b], g_sem)

        def body(step, carry):
            c0 = step * _NBUF
            for b in range(_NBUF):
                c = c0 + b
                pltpu.make_async_copy(
                    table_hbm.at[idx_v.at[c]], bufs[b], g_sem
                ).wait()
                pltpu.sync_copy(
                    bufs[b], out_hbm.at[pl.ds(base + c * _CHUNK, _CHUNK)]
                )

                @pl.when(c + _NBUF < n_chunks)
                def _():
                    pltpu.async_copy(
                        table_hbm.at[idx_v.at[c + _NBUF]], bufs[b], g_sem
                    )

            return carry

        lax.fori_loop(0, n_chunks // _NBUF, body, 0)

    return gather


def _mlp_body(flat_ref, w1_ref, b1_ref, w2_ref, b2_ref, out_ref):
    flat = flat_ref[...].astype(jnp.bfloat16)
    w1 = w1_ref[...].astype(jnp.bfloat16)
    h = jnp.tanh(
        jnp.dot(flat, w1, preferred_element_type=jnp.float32) + b1_ref[...]
    )
    w2 = w2_ref[...].astype(jnp.bfloat16)
    out_ref[...] = (
        jnp.dot(h.astype(jnp.bfloat16), w2, preferred_element_type=jnp.float32)
        + b2_ref[...]
    )


@functools.cache
def _make_mlp(batch, d_in, d_hidden, d_out, bm):
    grid = (batch // bm,)
    return pl.pallas_call(
        _mlp_body,
        grid=grid,
        in_specs=[
            pl.BlockSpec((bm, d_in), lambda i: (i, 0)),
            pl.BlockSpec((d_in, d_hidden), lambda i: (0, 0)),
            pl.BlockSpec((1, d_hidden), lambda i: (0, 0)),
            pl.BlockSpec((d_hidden, d_out), lambda i: (0, 0)),
            pl.BlockSpec((1, d_out), lambda i: (0, 0)),
        ],
        out_specs=pl.BlockSpec((bm, d_out), lambda i: (i, 0)),
        out_shape=jax.ShapeDtypeStruct((batch, d_out), jnp.float32),
    )


def kernel(x, table, W1, b1, W2, b2):
    batch, window = x.shape
    vocab, emb = table.shape
    hidden = W1.shape[1]
    n_rows = batch * window
    tbl = jnp.concatenate([table, jnp.zeros_like(table)], axis=1)
    idx = x.astype(jnp.int32).reshape(_NW, n_rows // (_NW * _CHUNK), _CHUNK)
    gathered = _make_gather(n_rows)(tbl, idx)
    flat = gathered.reshape(batch, window * _PW)
    w1p = jnp.concatenate(
        [
            W1.reshape(window, emb, hidden),
            jnp.zeros((window, _PW - emb, hidden), W1.dtype),
        ],
        axis=1,
    ).reshape(window * _PW, hidden)
    mlp = _make_mlp(batch, window * _PW, hidden, W2.shape[1], 2048)
    return mlp(flat, w1p, b1.reshape(1, -1), W2, b2.reshape(1, -1))


# speedup vs baseline: 6.2702x; 1.0764x over previous
import functools

import jax
import jax.numpy as jnp
from jax import lax
from jax.experimental import pallas as pl
from jax.experimental.pallas import tpu as pltpu
from jax.experimental.pallas import tpu_sc as plsc

_NC = 2
_NS = 16
_NW = _NC * _NS
_CHUNK = 128
_NBUF = 4
_PW = 128


@functools.cache
def _make_gather(n_rows):
    assert n_rows % (_NW * _CHUNK) == 0
    n_chunks = n_rows // (_NW * _CHUNK)
    per_w = n_chunks * _CHUNK
    mesh = plsc.VectorSubcoreMesh(core_axis_name="c", subcore_axis_name="s")

    @functools.partial(
        pl.kernel,
        out_type=jax.ShapeDtypeStruct((n_rows, _PW), jnp.float32),
        mesh=mesh,
        scratch_types=[
            pltpu.VMEM((n_chunks, _CHUNK), jnp.int32),
            *[pltpu.VMEM((_CHUNK, _PW), jnp.float32) for _ in range(_NBUF)],
            pltpu.SemaphoreType.DMA,
        ],
        compiler_params=pltpu.CompilerParams(
            use_tc_tiling_on_sc=False, needs_layout_passes=False
        ),
    )
    def gather(table_hbm, idx_hbm, out_hbm, idx_v, *rest):
        bufs, g_sem = rest[:_NBUF], rest[_NBUF]
        wid = lax.axis_index("s") * _NC + lax.axis_index("c")
        pltpu.sync_copy(idx_hbm.at[wid], idx_v)
        base = wid * per_w

        for b in range(_NBUF):
            pltpu.async_copy(table_hbm.at[idx_v.at[b]], bufs[b], g_sem)

        def body(step, carry):
            c0 = step * _NBUF
            for b in range(_NBUF):
                c = c0 + b
                pltpu.make_async_copy(
                    table_hbm.at[idx_v.at[c]], bufs[b], g_sem
                ).wait()
                pltpu.sync_copy(
                    bufs[b], out_hbm.at[pl.ds(base + c * _CHUNK, _CHUNK)]
                )

                @pl.when(c + _NBUF < n_chunks)
                def _():
                    pltpu.async_copy(
                        table_hbm.at[idx_v.at[c + _NBUF]], bufs[b], g_sem
                    )

            return carry

        lax.fori_loop(0, n_chunks // _NBUF, body, 0)

    return gather


def _mlp_body(*refs):
    *flat_refs, w1_ref, b1_ref, w2_ref, b2_ref, out_ref = refs
    window = len(flat_refs)
    w1 = w1_ref[...].astype(jnp.bfloat16)
    acc = b1_ref[...]
    for w in range(window):
        acc += jnp.dot(
            flat_refs[w][...].astype(jnp.bfloat16),
            w1[w * _PW : (w + 1) * _PW, :],
            preferred_element_type=jnp.float32,
        )
    h = jnp.tanh(acc)
    w2 = w2_ref[...].astype(jnp.bfloat16)
    out_ref[...] = (
        jnp.dot(h.astype(jnp.bfloat16), w2, preferred_element_type=jnp.float32)
        + b2_ref[...]
    )


@functools.cache
def _make_mlp(batch, window, d_hidden, d_out, bm):
    grid = (batch // bm,)
    nblk = batch // bm
    return pl.pallas_call(
        _mlp_body,
        grid=grid,
        in_specs=[
            *[
                pl.BlockSpec((bm, _PW), lambda i, w=w: (i + w * nblk, 0))
                for w in range(window)
            ],
            pl.BlockSpec((window * _PW, d_hidden), lambda i: (0, 0)),
            pl.BlockSpec((1, d_hidden), lambda i: (0, 0)),
            pl.BlockSpec((d_hidden, d_out), lambda i: (0, 0)),
            pl.BlockSpec((1, d_out), lambda i: (0, 0)),
        ],
        out_specs=pl.BlockSpec((bm, d_out), lambda i: (i, 0)),
        out_shape=jax.ShapeDtypeStruct((batch, d_out), jnp.float32),
    )


def kernel(x, table, W1, b1, W2, b2):
    batch, window = x.shape
    vocab, emb = table.shape
    hidden = W1.shape[1]
    n_rows = batch * window
    tbl = jnp.pad(table, ((0, 0), (0, _PW - emb)))
    idx = x.T.astype(jnp.int32).reshape(_NW, n_rows // (_NW * _CHUNK), _CHUNK)
    gathered = _make_gather(n_rows)(tbl, idx)
    w1p = jnp.concatenate(
        [
            W1.reshape(window, emb, hidden),
            jnp.zeros((window, _PW - emb, hidden), W1.dtype),
        ],
        axis=1,
    ).reshape(window * _PW, hidden)
    mlp = _make_mlp(batch, window, hidden, W2.shape[1], 2048)
    return mlp(
        *([gathered] * window),
        w1p,
        b1.reshape(1, -1),
        W2,
        b2.reshape(1, -1),
    )
